# initial kernel scaffold (unmeasured)
import jax
import jax.numpy as jnp
from jax import lax
from jax.experimental import pallas as pl
from jax.experimental.pallas import tpu as pltpu

N_DEV = 8


def kernel(x, w_mat):
    m_per, k = x.shape
    _, n = w_mat.shape
    n_per = n // N_DEV

    def body(x_ref, w_hbm, out_ref, w_buf, send_buf, w_sems, send_sems, recv_sems):
        my = lax.axis_index("i")

        bar = pltpu.get_barrier_semaphore()
        for p in range(N_DEV):
            pl.semaphore_signal(
                bar, inc=1, device_id=(p,), device_id_type=pl.DeviceIdType.MESH
            )
        pl.semaphore_wait(bar, N_DEV)

        for off in range(N_DEV):
            slot = off % 2
            j = lax.rem(my + off, N_DEV)
            cp = pltpu.make_async_copy(
                w_hbm.at[:, pl.ds(j * n_per, n_per)],
                w_buf.at[slot],
                w_sems.at[slot],
            )
            cp.start()
            cp.wait()
            yb = jnp.dot(
                x_ref[...], w_buf[slot], preferred_element_type=jnp.float32
            ).astype(jnp.bfloat16)
            if off == 0:
                out_ref[pl.ds(my * m_per, m_per), :] = yb
            else:
                send_buf[slot, :, :] = yb
                rdma = pltpu.make_async_remote_copy(
                    src_ref=send_buf.at[slot],
                    dst_ref=out_ref.at[pl.ds(my * m_per, m_per), :],
                    send_sem=send_sems.at[slot],
                    recv_sem=recv_sems.at[off - 1],
                    device_id=(j,),
                    device_id_type=pl.DeviceIdType.MESH,
                )
                rdma.start()
                rdma.wait_send()

        for off in range(1, N_DEV):
            recv = pltpu.make_async_remote_copy(
                src_ref=send_buf.at[0],
                dst_ref=out_ref.at[pl.ds(0, m_per), :],
                send_sem=send_sems.at[0],
                recv_sem=recv_sems.at[off - 1],
                device_id=(my,),
                device_id_type=pl.DeviceIdType.MESH,
            )
            recv.wait_recv()

    out_shape = jax.ShapeDtypeStruct((N_DEV * m_per, n_per), jnp.bfloat16)
    return pl.pallas_call(
        body,
        out_shape=out_shape,
        in_specs=[
            pl.BlockSpec(memory_space=pltpu.VMEM),
            pl.BlockSpec(memory_space=pltpu.ANY),
        ],
        out_specs=pl.BlockSpec(memory_space=pltpu.VMEM),
        scratch_shapes=[
            pltpu.VMEM((2, k, n_per), jnp.bfloat16),
            pltpu.VMEM((2, m_per, n_per), jnp.bfloat16),
            pltpu.SemaphoreType.DMA((2,)),
            pltpu.SemaphoreType.DMA((2,)),
            pltpu.SemaphoreType.DMA((N_DEV - 1,)),
        ],
        compiler_params=pltpu.CompilerParams(collective_id=0),
    )(x, w_mat)


# baseline (device time: 203773 ns/iter reference)
import jax
import jax.numpy as jnp
from jax import lax
from jax.experimental import pallas as pl
from jax.experimental.pallas import tpu as pltpu

N_DEV = 8
N_HALF = 2


def kernel(x, w_mat):
    m_per, k = x.shape
    _, n = w_mat.shape
    n_per = n // N_DEV
    n_half = n_per // N_HALF
    n_steps = N_DEV * N_HALF

    x = x.astype(jnp.bfloat16)

    def body(x_ref, w_hbm, out_ref, w_stage, send_buf, w_sems, send_sems, recv_sems):
        my = lax.axis_index("i")

        bar = pltpu.get_barrier_semaphore()
        for p in range(N_DEV):
            pl.semaphore_signal(
                bar, inc=1, device_id=(p,), device_id_type=pl.DeviceIdType.MESH
            )
        pl.semaphore_wait(bar, N_DEV)

        def w_dma(s):
            blk = s // 2
            col = lax.rem(my + blk, N_DEV) * n_per + (s % 2) * n_half
            return pltpu.make_async_copy(
                w_hbm.at[:, pl.ds(col, n_half)],
                w_stage.at[s % 2],
                w_sems.at[s % 2],
            )

        w_dma(jnp.int32(0)).start()

        def step(s, carry):
            blk = s // 2
            half = s % 2
            w_dma(s).wait()

            @pl.when(s + 1 < n_steps)
            def _():
                w_dma(s + 1).start()

            yh = jnp.dot(
                x_ref[...],
                w_stage[s % 2].astype(jnp.bfloat16),
                preferred_element_type=jnp.float32,
            ).astype(jnp.bfloat16)

            @pl.when(blk == 0)
            def _():
                out_ref[pl.ds(my * m_per, m_per), pl.ds(half * n_half, n_half)] = yh

            @pl.when(blk > 0)
            def _():
                send_buf[blk % 2, :, pl.ds(half * n_half, n_half)] = yh

            @pl.when(jnp.logical_and(blk > 0, half == 1))
            def _():
                rdma = pltpu.make_async_remote_copy(
                    src_ref=send_buf.at[blk % 2],
                    dst_ref=out_ref.at[pl.ds(my * m_per, m_per), :],
                    send_sem=send_sems.at[blk % 2],
                    recv_sem=recv_sems.at[blk - 1],
                    device_id=(lax.rem(my + blk, N_DEV),),
                    device_id_type=pl.DeviceIdType.MESH,
                )
                rdma.start()
                rdma.wait_send()

            return carry

        lax.fori_loop(0, n_steps, step, 0)

        for blk in range(1, N_DEV):
            recv = pltpu.make_async_remote_copy(
                src_ref=send_buf.at[0],
                dst_ref=out_ref.at[pl.ds(0, m_per), :],
                send_sem=send_sems.at[0],
                recv_sem=recv_sems.at[blk - 1],
                device_id=(my,),
                device_id_type=pl.DeviceIdType.MESH,
            )
            recv.wait_recv()

    out_shape = jax.ShapeDtypeStruct((N_DEV * m_per, n_per), jnp.bfloat16)
    return pl.pallas_call(
        body,
        out_shape=out_shape,
        in_specs=[
            pl.BlockSpec(memory_space=pltpu.VMEM),
            pl.BlockSpec(memory_space=pl.ANY),
        ],
        out_specs=pl.BlockSpec(memory_space=pltpu.VMEM),
        scratch_shapes=[
            pltpu.VMEM((2, k, n_half), jnp.float32),
            pltpu.VMEM((2, m_per, n_per), jnp.bfloat16),
            pltpu.SemaphoreType.DMA((2,)),
            pltpu.SemaphoreType.DMA((2,)),
            pltpu.SemaphoreType.DMA((N_DEV - 1,)),
        ],
        compiler_params=pltpu.CompilerParams(
            collective_id=0,
            vmem_limit_bytes=64 * 1024 * 1024,
        ),
    )(x, w_mat)


# device time: 144444 ns/iter; 1.4107x vs baseline; 1.4107x over previous
import jax
import jax.numpy as jnp
from jax import lax
from jax.experimental import pallas as pl
from jax.experimental.pallas import tpu as pltpu

N_DEV = 8
N_HALF = 2


def kernel(x, w_mat):
    m_per, k = x.shape
    _, n = w_mat.shape
    n_per = n // N_DEV
    n_half = n_per // N_HALF
    n_steps = N_DEV * N_HALF

    x = x.astype(jnp.bfloat16)

    def body(x_ref, w_hbm, out_ref, w_stage, send_buf, w_sems, send_sems, recv_sems):
        my = lax.axis_index("i")

        bar = pltpu.get_barrier_semaphore()
        for p in range(N_DEV):
            pl.semaphore_signal(
                bar, inc=1, device_id=(p,), device_id_type=pl.DeviceIdType.MESH
            )
        pl.semaphore_wait(bar, N_DEV)

        def tgt_dev(blk):
            return jnp.where(blk < N_DEV - 1, lax.rem(my + 1 + blk, N_DEV), my)

        def w_dma(s):
            col = tgt_dev(s // 2) * n_per + (s % 2) * n_half
            return pltpu.make_async_copy(
                w_hbm.at[:, pl.ds(col, n_half)],
                w_stage.at[s % 2],
                w_sems.at[s % 2],
            )

        def send_desc(sslot, blk):
            return pltpu.make_async_remote_copy(
                src_ref=send_buf.at[sslot],
                dst_ref=out_ref.at[pl.ds(my * m_per, m_per), :],
                send_sem=send_sems.at[sslot],
                recv_sem=recv_sems.at[blk],
                device_id=(tgt_dev(blk),),
                device_id_type=pl.DeviceIdType.MESH,
            )

        w_dma(jnp.int32(0)).start()

        def step(s, carry):
            blk = s // 2
            half = s % 2
            w_dma(s).wait()

            @pl.when(s + 1 < n_steps)
            def _():
                w_dma(s + 1).start()

            yh = jnp.dot(
                x_ref[...],
                w_stage[s % 2].astype(jnp.bfloat16),
                preferred_element_type=jnp.float32,
            ).astype(jnp.bfloat16)

            @pl.when(blk == N_DEV - 1)
            def _():
                out_ref[pl.ds(my * m_per, m_per), pl.ds(half * n_half, n_half)] = yh

            @pl.when(blk < N_DEV - 1)
            def _():
                @pl.when(jnp.logical_and(half == 0, blk >= 2))
                def _():
                    send_desc(blk % 2, blk - 2).wait_send()

                send_buf[blk % 2, :, pl.ds(half * n_half, n_half)] = yh

                @pl.when(half == 1)
                def _():
                    send_desc(blk % 2, blk).start()

            return carry

        lax.fori_loop(0, n_steps, step, 0)

        send_desc(1, jnp.int32(N_DEV - 3)).wait_send()
        send_desc(0, jnp.int32(N_DEV - 2)).wait_send()

        for blk in range(N_DEV - 1):
            recv = pltpu.make_async_remote_copy(
                src_ref=send_buf.at[0],
                dst_ref=out_ref.at[pl.ds(0, m_per), :],
                send_sem=send_sems.at[0],
                recv_sem=recv_sems.at[blk],
                device_id=(my,),
                device_id_type=pl.DeviceIdType.MESH,
            )
            recv.wait_recv()

    out_shape = jax.ShapeDtypeStruct((N_DEV * m_per, n_per), jnp.bfloat16)
    return pl.pallas_call(
        body,
        out_shape=out_shape,
        in_specs=[
            pl.BlockSpec(memory_space=pltpu.VMEM),
            pl.BlockSpec(memory_space=pl.ANY),
        ],
        out_specs=pl.BlockSpec(memory_space=pltpu.VMEM),
        scratch_shapes=[
            pltpu.VMEM((2, k, n_half), jnp.float32),
            pltpu.VMEM((2, m_per, n_per), jnp.bfloat16),
            pltpu.SemaphoreType.DMA((2,)),
            pltpu.SemaphoreType.DMA((2,)),
            pltpu.SemaphoreType.DMA((N_DEV - 1,)),
        ],
        compiler_params=pltpu.CompilerParams(
            collective_id=0,
            vmem_limit_bytes=64 * 1024 * 1024,
        ),
    )(x, w_mat)


# device time: 105638 ns/iter; 1.9290x vs baseline; 1.3673x over previous
import os

import jax
import jax.numpy as jnp
from jax import lax
from jax.experimental import pallas as pl
from jax.experimental.pallas import tpu as pltpu

N_DEV = 8
N_HALF = 2

_VARIANT = os.environ.get("KERNEL_VARIANT", "full")


def kernel(x, w_mat):
    m_per, k = x.shape
    _, n = w_mat.shape
    n_per = n // N_DEV
    n_half = n_per // N_HALF
    n_steps = N_DEV * N_HALF

    x = x.astype(jnp.bfloat16)

    def body(x_ref, w_hbm, out_ref, w_stage, send_buf, w_sems, send_sems, recv_sems):
        my = lax.axis_index("i")

        if _VARIANT != "nocomm":
            bar = pltpu.get_barrier_semaphore()
            for p in range(N_DEV):
                pl.semaphore_signal(
                    bar, inc=1, device_id=(p,), device_id_type=pl.DeviceIdType.MESH
                )
            pl.semaphore_wait(bar, N_DEV)

        def tgt_dev(blk):
            return jnp.where(blk < N_DEV - 1, lax.rem(my + 1 + blk, N_DEV), my)

        def w_dma(s):
            col = tgt_dev(s // 2) * n_per + (s % 2) * n_half
            return pltpu.make_async_copy(
                w_hbm.at[:, pl.ds(col, n_half)],
                w_stage.at[s % 2],
                w_sems.at[s % 2],
            )

        def send_desc(sslot, blk):
            return pltpu.make_async_remote_copy(
                src_ref=send_buf.at[sslot],
                dst_ref=out_ref.at[pl.ds(my * m_per, m_per), :],
                send_sem=send_sems.at[sslot],
                recv_sem=recv_sems.at[blk],
                device_id=(tgt_dev(blk),),
                device_id_type=pl.DeviceIdType.MESH,
            )

        w_dma(jnp.int32(0)).start()

        def step(s, carry):
            blk = s // 2
            half = s % 2
            w_dma(s).wait()

            @pl.when(s + 1 < n_steps)
            def _():
                w_dma(s + 1).start()

            if _VARIANT != "nocompute":
                yh = jnp.dot(
                    x_ref[...],
                    w_stage[s % 2].astype(jnp.bfloat16),
                    preferred_element_type=jnp.float32,
                ).astype(jnp.bfloat16)
            else:
                yh = jnp.full(
                    (m_per, n_half), w_stage[s % 2, 0, 0], dtype=jnp.float32
                ).astype(jnp.bfloat16)

            @pl.when(blk == N_DEV - 1)
            def _():
                out_ref[pl.ds(my * m_per, m_per), pl.ds(half * n_half, n_half)] = yh

            @pl.when(blk < N_DEV - 1)
            def _():
                if _VARIANT != "nocomm":

                    @pl.when(jnp.logical_and(half == 0, blk >= 2))
                    def _():
                        send_desc(blk % 2, blk - 2).wait_send()

                send_buf[blk % 2, :, pl.ds(half * n_half, n_half)] = yh

                if _VARIANT != "nocomm":

                    @pl.when(half == 1)
                    def _():
                        send_desc(blk % 2, blk).start()

            return carry

        lax.fori_loop(0, n_steps, step, 0)

        if _VARIANT != "nocomm":
            send_desc(1, jnp.int32(N_DEV - 3)).wait_send()
            send_desc(0, jnp.int32(N_DEV - 2)).wait_send()

            for blk in range(N_DEV - 1):
                recv = pltpu.make_async_remote_copy(
                    src_ref=send_buf.at[0],
                    dst_ref=out_ref.at[pl.ds(0, m_per), :],
                    send_sem=send_sems.at[0],
                    recv_sem=recv_sems.at[blk],
                    device_id=(my,),
                    device_id_type=pl.DeviceIdType.MESH,
                )
                recv.wait_recv()

    out_shape = jax.ShapeDtypeStruct((N_DEV * m_per, n_per), jnp.bfloat16)
    return pl.pallas_call(
        body,
        out_shape=out_shape,
        in_specs=[
            pl.BlockSpec(memory_space=pltpu.VMEM),
            pl.BlockSpec(memory_space=pl.ANY),
        ],
        out_specs=pl.BlockSpec(memory_space=pltpu.VMEM),
        scratch_shapes=[
            pltpu.VMEM((2, k, n_half), jnp.float32),
            pltpu.VMEM((2, m_per, n_per), jnp.bfloat16),
            pltpu.SemaphoreType.DMA((2,)),
            pltpu.SemaphoreType.DMA((2,)),
            pltpu.SemaphoreType.DMA((N_DEV - 1,)),
        ],
        compiler_params=pltpu.CompilerParams(
            collective_id=None if _VARIANT == "nocomm" else 0,
            vmem_limit_bytes=64 * 1024 * 1024,
        ),
    )(x, w_mat)
